# baseline (speedup 1.0000x reference)
import functools

import jax
import jax.numpy as jnp
from jax import lax
from jax.experimental import pallas as pl
from jax.experimental.pallas import tpu as pltpu
from jax.experimental.pallas import tpu_sc as plsc

_D = 4096
_B = 257
_NC = 2
_NS = 16
_NW = _NC * _NS
_RPW = 8

_mesh = plsc.VectorSubcoreMesh(core_axis_name="c", subcore_axis_name="s")


@functools.partial(
    pl.kernel,
    out_type=[
        jax.ShapeDtypeStruct((1, _D), jnp.float32),
        jax.ShapeDtypeStruct((_B - 1, _D), jnp.float32),
    ],
    mesh=_mesh,
    compiler_params=pltpu.CompilerParams(use_tc_tiling_on_sc=False),
    scratch_types=[
        pltpu.VMEM((_RPW,), jnp.int32),
        pltpu.VMEM((_RPW, _D), jnp.float32),
        pltpu.VMEM((1,), jnp.int32),
        pltpu.VMEM((1, _D), jnp.float32),
        pltpu.SemaphoreType.DMA,
        pltpu.SemaphoreType.DMA,
        pltpu.SemaphoreType.DMA,
    ],
)
def _sc_gather(table_hbm, idx_hbm, pc_hbm, fields_hbm,
               idx_v, rows_v, idxe_v, row_e,
               sem_g, sem_e, sem_s):
    wid = lax.axis_index("s") * _NC + lax.axis_index("c")
    base = wid * _RPW
    is_w0 = wid == 0
    is_we = wid == 1

    pltpu.sync_copy(idx_hbm.at[pl.ds(base, _RPW)], idx_v)
    g = pltpu.async_copy(table_hbm.at[idx_v], rows_v, sem_g)

    @pl.when(is_we)
    def _():
        pltpu.sync_copy(idx_hbm.at[pl.ds(_B - 1, 1)], idxe_v)
        pltpu.async_copy(table_hbm.at[idxe_v], row_e, sem_e)

    g.wait()

    @pl.when(is_w0)
    def _():
        sp = pltpu.async_copy(rows_v.at[pl.ds(0, 1)], pc_hbm, sem_s)
        pltpu.async_copy(rows_v.at[pl.ds(1, _RPW - 1)],
                         fields_hbm.at[pl.ds(0, _RPW - 1)], sem_s).wait()
        sp.wait()

    @pl.when(jnp.logical_not(is_w0))
    def _():
        pltpu.async_copy(rows_v, fields_hbm.at[pl.ds(base - 1, _RPW)],
                         sem_s).wait()

    @pl.when(is_we)
    def _():
        pltpu.make_async_copy(table_hbm.at[idxe_v], row_e, sem_e).wait()
        pltpu.async_copy(row_e, fields_hbm.at[pl.ds(_B - 2, 1)], sem_e).wait()


def kernel(hidden_state, schema_indices):
    table = hidden_state[0]
    pc_emb, field_embs = _sc_gather(table, schema_indices)
    return (pc_emb, field_embs)


# speedup vs baseline: 5.1071x; 5.1071x over previous
import functools

import jax
import jax.numpy as jnp
from jax import lax
from jax.experimental import pallas as pl
from jax.experimental.pallas import tpu as pltpu
from jax.experimental.pallas import tpu_sc as plsc

_D = 4096
_B = 257
_NC = 2
_NS = 16
_NW = _NC * _NS
_RPW = 8

_mesh = plsc.VectorSubcoreMesh(core_axis_name="c", subcore_axis_name="s")


@functools.partial(
    pl.kernel,
    out_type=[
        jax.ShapeDtypeStruct((1, _D), jnp.float32),
        jax.ShapeDtypeStruct((_B - 1, _D), jnp.float32),
    ],
    mesh=_mesh,
    compiler_params=pltpu.CompilerParams(needs_layout_passes=False),
    scratch_types=[
        pltpu.VMEM((16,), jnp.int32),
        pltpu.VMEM((16,), jnp.int32),
        pltpu.VMEM((_RPW, _D), jnp.float32),
        pltpu.VMEM((1,), jnp.int32),
        pltpu.VMEM((1, _D), jnp.float32),
        pltpu.SemaphoreType.DMA,
        pltpu.SemaphoreType.DMA,
        pltpu.SemaphoreType.DMA,
    ],
)
def _sc_gather(table_hbm, idx_hbm, pc_hbm, fields_hbm,
               idx_v16, idx_s, rows_v, idxp_v, row_pc,
               sem_g, sem_p, sem_s):
    wid = lax.axis_index("s") * _NC + lax.axis_index("c")
    base = wid * _RPW
    is_w0 = wid == 0
    is_wlast = wid == _NW - 1

    @pl.when(jnp.logical_not(is_wlast))
    def _():
        pltpu.sync_copy(idx_hbm.at[pl.ds(base, 16)], idx_v16)

    @pl.when(is_wlast)
    def _():
        pltpu.sync_copy(idx_hbm.at[pl.ds(_B - 9, 8)], idx_v16.at[pl.ds(0, 8)])
        pltpu.sync_copy(idx_hbm.at[pl.ds(_B - 1, 1)], idx_v16.at[pl.ds(8, 1)])

    shift = jnp.minimum(lax.iota(jnp.int32, 16) + 1, 15)
    idx_s[...] = plsc.load_gather(idx_v16, [shift])
    g = pltpu.async_copy(table_hbm.at[idx_s.at[pl.ds(0, _RPW)]], rows_v, sem_g)

    @pl.when(is_w0)
    def _():
        pltpu.sync_copy(idx_hbm.at[pl.ds(0, 1)], idxp_v)
        pltpu.async_copy(table_hbm.at[idxp_v], row_pc, sem_p)

    g.wait()
    s = pltpu.async_copy(rows_v, fields_hbm.at[pl.ds(base, _RPW)], sem_s)

    @pl.when(is_w0)
    def _():
        pltpu.make_async_copy(table_hbm.at[idxp_v], row_pc, sem_p).wait()
        pltpu.async_copy(row_pc, pc_hbm, sem_p).wait()

    s.wait()


def kernel(hidden_state, schema_indices):
    table = hidden_state[0]
    pc_emb, field_embs = _sc_gather(table, schema_indices)
    return (pc_emb, field_embs)


# speedup vs baseline: 5.1366x; 1.0058x over previous
import functools

import jax
import jax.numpy as jnp
from jax import lax
from jax.experimental import pallas as pl
from jax.experimental.pallas import tpu as pltpu
from jax.experimental.pallas import tpu_sc as plsc

_D = 4096
_B = 257
_NC = 2
_NS = 16
_NW = _NC * _NS
_RPW = 8
_CHUNK = 4

_mesh = plsc.VectorSubcoreMesh(core_axis_name="c", subcore_axis_name="s")


@functools.partial(
    pl.kernel,
    out_type=[
        jax.ShapeDtypeStruct((1, _D), jnp.float32),
        jax.ShapeDtypeStruct((_B - 1, _D), jnp.float32),
    ],
    mesh=_mesh,
    compiler_params=pltpu.CompilerParams(needs_layout_passes=False),
    scratch_types=[
        pltpu.VMEM((16,), jnp.int32),
        pltpu.VMEM((16,), jnp.int32),
        pltpu.VMEM((16,), jnp.int32),
        pltpu.VMEM((_CHUNK, _D), jnp.float32),
        pltpu.VMEM((_CHUNK, _D), jnp.float32),
        pltpu.VMEM((1,), jnp.int32),
        pltpu.VMEM((1, _D), jnp.float32),
        pltpu.SemaphoreType.DMA,
        pltpu.SemaphoreType.DMA,
        pltpu.SemaphoreType.DMA,
        pltpu.SemaphoreType.DMA,
    ],
)
def _sc_gather(table_hbm, idx_hbm, pc_hbm, fields_hbm,
               idx_v16, idx_sa, idx_sb, rows_a, rows_b, idxp_v, row_pc,
               sem_a, sem_b, sem_p, sem_s):
    wid = lax.axis_index("s") * _NC + lax.axis_index("c")
    base = wid * _RPW
    is_w0 = wid == 0
    is_wlast = wid == _NW - 1

    @pl.when(jnp.logical_not(is_wlast))
    def _():
        pltpu.sync_copy(idx_hbm.at[pl.ds(base, 16)], idx_v16)

    @pl.when(is_wlast)
    def _():
        pltpu.sync_copy(idx_hbm.at[pl.ds(_B - 9, 8)], idx_v16.at[pl.ds(0, 8)])
        pltpu.sync_copy(idx_hbm.at[pl.ds(_B - 1, 1)], idx_v16.at[pl.ds(8, 1)])

    lanes = lax.iota(jnp.int32, 16)
    idx_sa[...] = plsc.load_gather(idx_v16, [jnp.minimum(lanes + 1, 15)])
    idx_sb[...] = plsc.load_gather(idx_v16, [jnp.minimum(lanes + 1 + _CHUNK, 15)])
    ga = pltpu.async_copy(table_hbm.at[idx_sa.at[pl.ds(0, _CHUNK)]], rows_a, sem_a)
    gb = pltpu.async_copy(table_hbm.at[idx_sb.at[pl.ds(0, _CHUNK)]], rows_b, sem_b)

    @pl.when(is_w0)
    def _():
        pltpu.sync_copy(idx_hbm.at[pl.ds(0, 1)], idxp_v)
        pltpu.async_copy(table_hbm.at[idxp_v], row_pc, sem_p)

    ga.wait()
    sa = pltpu.async_copy(rows_a, fields_hbm.at[pl.ds(base, _CHUNK)], sem_s)
    gb.wait()
    sb = pltpu.async_copy(rows_b, fields_hbm.at[pl.ds(base + _CHUNK, _CHUNK)], sem_s)

    @pl.when(is_w0)
    def _():
        pltpu.make_async_copy(table_hbm.at[idxp_v], row_pc, sem_p).wait()
        pltpu.async_copy(row_pc, pc_hbm, sem_p).wait()

    sa.wait()
    sb.wait()


def kernel(hidden_state, schema_indices):
    table = hidden_state[0]
    pc_emb, field_embs = _sc_gather(table, schema_indices)
    return (pc_emb, field_embs)
